# baseline (speedup 1.0000x reference)
import functools

import jax
import jax.numpy as jnp
from jax import lax
from jax.experimental import pallas as pl
from jax.experimental.pallas import tpu as pltpu
from jax.experimental.pallas import tpu_sc as plsc

_UPW = 64


def _sc_body(in_hbm, out_hbm, ib0, ib1, ob0, ob1, si0, si1, so0, so1):
    wid = lax.axis_index("s") * 2 + lax.axis_index("c")
    ibs = (ib0, ib1)
    obs = (ob0, ob1)
    sis = (si0, si1)
    sos = (so0, so1)

    lane = lax.iota(jnp.int32, 16)
    cw_g = [
        32 * g + 16 * (lane >> 3) + 8 * ((lane >> 2) & 1) for g in range(8)
    ]
    jpart = (lane >> 1) & 1
    zz_init = lane & 1

    def decode(u):
        uid = wid * _UPW + u
        return uid >> 10, (uid >> 4) & 63, uid & 15

    def in_cps(u, sb):
        b, h, wq = decode(u)
        cps = []
        for c in range(16):
            for hl in range(2):
                cps.append(
                    pltpu.make_async_copy(
                        in_hbm.at[b, c, 2 * h + hl, pl.ds(8 * wq, 8)],
                        ibs[sb].at[pl.ds((c * 2 + hl) * 8, 8)],
                        sis[sb],
                    )
                )
        return cps

    def out_cp(u, ws, p):
        b, h, wq = decode(u)
        return pltpu.make_async_copy(
            obs[p], out_hbm.at[b, h, wq * 4 + ws], sos[p]
        )

    def compute(u, sb, wait01):
        src = ibs[sb]
        for ws in range(4):
            p = ws & 1
            if ws >= 2 or wait01:
                out_cp(u, ws, p).wait()
            dst = obs[p]
            cwv = [cw_g[g] + (2 * ws) + jpart for g in range(8)]

            @plsc.parallel_loop(0, 64, carry=zz_init, unroll=2)
            def z_loop(z, zzv):
                for g in range(8):
                    v = plsc.load_gather(src, [cwv[g], zzv])
                    dst[z, pl.ds(16 * g, 16)] = v
                return zzv + 2

            out_cp(u, ws, p).start()

    for cp in in_cps(0, 0):
        cp.start()

    def pair(up, carry):
        for sb in range(2):
            u = 2 * up + sb
            for cp in in_cps(u, sb):
                cp.wait()

            @pl.when(u < _UPW - 1)
            def _():
                for cp in in_cps(u + 1, 1 - sb):
                    cp.start()

            if sb == 0:

                @pl.when(up > 0)
                def _():
                    out_cp(u, 0, 0).wait()
                    out_cp(u, 1, 1).wait()

                compute(u, sb, False)
            else:
                compute(u, sb, True)
        return carry

    lax.fori_loop(0, _UPW // 2, pair, 0)

    out_cp(_UPW - 1, 2, 0).wait()
    out_cp(_UPW - 1, 3, 1).wait()


def kernel(x):
    B, C, H, W, Z = x.shape
    mesh = plsc.VectorSubcoreMesh(core_axis_name="c", subcore_axis_name="s")
    run = functools.partial(
        pl.kernel,
        mesh=mesh,
        out_type=jax.ShapeDtypeStruct(
            (B, H // 2, W // 2, Z // 2, C * 8), x.dtype
        ),
        scratch_types=[
            pltpu.VMEM((256, 128), jnp.float32),
            pltpu.VMEM((256, 128), jnp.float32),
            pltpu.VMEM((64, 128), jnp.float32),
            pltpu.VMEM((64, 128), jnp.float32),
            pltpu.SemaphoreType.DMA,
            pltpu.SemaphoreType.DMA,
            pltpu.SemaphoreType.DMA,
            pltpu.SemaphoreType.DMA,
        ],
        compiler_params=pltpu.CompilerParams(needs_layout_passes=False),
    )(_sc_body)
    out = run(x)
    return jnp.transpose(out, (0, 4, 1, 2, 3))


# speedup vs baseline: 1.0587x; 1.0587x over previous
import functools

import jax
import jax.numpy as jnp
from jax import lax
from jax.experimental import pallas as pl
from jax.experimental.pallas import tpu as pltpu
from jax.experimental.pallas import tpu_sc as plsc

_UPW = 64


def _sc_body(in_hbm, out_hbm, ib0, ib1, ob0, ob1, si0, si1, so0, so1):
    wid = lax.axis_index("s") * 2 + lax.axis_index("c")
    ibs = (ib0, ib1)
    obs = (ob0, ob1)
    sis = (si0, si1)
    sos = (so0, so1)

    lane = lax.iota(jnp.int32, 16)
    cw_g = [
        32 * g + 16 * (lane >> 3) + 8 * ((lane >> 2) & 1) for g in range(8)
    ]
    jpart = (lane >> 1) & 1
    zz_init = lane & 1

    def decode(u):
        uid = wid * _UPW + u
        return uid >> 10, (uid >> 4) & 63, uid & 15

    def in_cps(u, sb):
        b, h, wq = decode(u)
        cps = []
        for c in range(16):
            for hl in range(2):
                cps.append(
                    pltpu.make_async_copy(
                        in_hbm.at[b, c, 2 * h + hl, pl.ds(8 * wq, 8)],
                        ibs[sb].at[pl.ds((c * 2 + hl) * 8, 8)],
                        sis[sb],
                    )
                )
        return cps

    def out_cp(u, ws, p):
        b, h, wq = decode(u)
        return pltpu.make_async_copy(
            obs[p], out_hbm.at[b, h, wq * 4 + ws], sos[p]
        )

    def compute(u, sb, wait01):
        src = ibs[sb]
        for ws in range(4):
            p = ws & 1
            if ws >= 2 or wait01:
                out_cp(u, ws, p).wait()
            dst = obs[p]
            cwv = [cw_g[g] + (2 * ws) + jpart for g in range(8)]

            @plsc.parallel_loop(0, 64, carry=zz_init)
            def z_loop(z, zzv):
                for g in range(8):
                    v = plsc.load_gather(src, [cwv[g], zzv])
                    dst[z, pl.ds(16 * g, 16)] = v
                return zzv + 2

            out_cp(u, ws, p).start()

    for cp in in_cps(0, 0):
        cp.start()

    def pair(up, carry):
        for sb in range(2):
            u = 2 * up + sb
            for cp in in_cps(u, sb):
                cp.wait()

            @pl.when(u < _UPW - 1)
            def _():
                for cp in in_cps(u + 1, 1 - sb):
                    cp.start()

            if sb == 0:

                @pl.when(up > 0)
                def _():
                    out_cp(u, 0, 0).wait()
                    out_cp(u, 1, 1).wait()

                compute(u, sb, False)
            else:
                compute(u, sb, True)
        return carry

    lax.fori_loop(0, _UPW // 2, pair, 0)

    out_cp(_UPW - 1, 2, 0).wait()
    out_cp(_UPW - 1, 3, 1).wait()


def kernel(x):
    B, C, H, W, Z = x.shape
    mesh = plsc.VectorSubcoreMesh(core_axis_name="c", subcore_axis_name="s")
    run = functools.partial(
        pl.kernel,
        mesh=mesh,
        out_type=jax.ShapeDtypeStruct(
            (B, H // 2, W // 2, Z // 2, C * 8), x.dtype
        ),
        scratch_types=[
            pltpu.VMEM((256, 128), jnp.float32),
            pltpu.VMEM((256, 128), jnp.float32),
            pltpu.VMEM((64, 128), jnp.float32),
            pltpu.VMEM((64, 128), jnp.float32),
            pltpu.SemaphoreType.DMA,
            pltpu.SemaphoreType.DMA,
            pltpu.SemaphoreType.DMA,
            pltpu.SemaphoreType.DMA,
        ],
        compiler_params=pltpu.CompilerParams(needs_layout_passes=False),
    )(_sc_body)
    out = run(x)
    return jnp.transpose(out, (0, 4, 1, 2, 3))
